# Initial kernel scaffold; baseline (speedup 1.0000x reference)
import jax
import jax.numpy as jnp
from jax.experimental import pallas as pl


def kernel(x):
    raise NotImplementedError("write your pallas kernel here")


# speedup vs baseline: 892.0765x; 892.0765x over previous
import math

import jax
import jax.numpy as jnp
import numpy as np
from jax.experimental import pallas as pl
from jax.experimental.pallas import tpu as pltpu

_N = 65536
_R = 256

_k = np.arange(_R, dtype=np.float64)
_F = np.exp(-2j * np.pi * np.outer(_k, _k) / _R)
_T = np.exp(-2j * np.pi * np.outer(_k, _k) / _N)
_FR = jnp.asarray(_F.real, jnp.float32)
_FI = jnp.asarray(_F.imag, jnp.float32)
_TR = jnp.asarray(_T.real, jnp.float32)
_TI = jnp.asarray(_T.imag, jnp.float32)

_PREC = jax.lax.Precision.HIGHEST


def _mm(a, b, dn):
    return jax.lax.dot_general(a, b, dimension_numbers=dn,
                               precision=_PREC,
                               preferred_element_type=jnp.float32)


def _fft_body(xr_ref, xi_ref, fr_ref, fi_ref, tr_ref, ti_ref,
              or_ref, oi_ref):
    xr = xr_ref[...]
    xi = xi_ref[...]
    fr = fr_ref[...]
    fi = fi_ref[...]
    dn_nn = (((1,), (0,)), ((), ()))
    dr = _mm(fr, xr, dn_nn) - _mm(fi, xi, dn_nn)
    di = _mm(fr, xi, dn_nn) + _mm(fi, xr, dn_nn)
    tr = tr_ref[...]
    ti = ti_ref[...]
    cr = dr * tr - di * ti
    ci = dr * ti + di * tr
    dn_nt = (((1,), (1,)), ((), ()))
    or_ref[...] = _mm(fr, cr, dn_nt) - _mm(fi, ci, dn_nt)
    oi_ref[...] = _mm(fr, ci, dn_nt) + _mm(fi, cr, dn_nt)


def kernel(x):
    x2 = x.reshape(_R, _R, 2)
    xr = x2[:, :, 0]
    xi = x2[:, :, 1]
    vspec = pl.BlockSpec(memory_space=pltpu.VMEM)
    yr, yi = pl.pallas_call(
        _fft_body,
        in_specs=[vspec] * 6,
        out_specs=[vspec, vspec],
        out_shape=[jax.ShapeDtypeStruct((_R, _R), jnp.float32)] * 2,
    )(xr, xi, _FR, _FI, _TR, _TI)
    return jnp.stack((yr.reshape(_N), yi.reshape(_N)), axis=-1)
